# baseline (speedup 1.0000x reference)
import functools

import jax
import jax.numpy as jnp
from jax import lax
from jax.experimental import pallas as pl
from jax.experimental.pallas import tpu as pltpu
from jax.experimental.pallas import tpu_sc as plsc

N = 10000
F_IN = 128
H_DIM = 32
B = 100

CHUNK = 640
TAIL = N - 15 * CHUNK
B_PAD = 112
ACC = 16 * B_PAD
BLK = 1000


def _dense_body(x_ref, wz0_ref, wz1_ref, wh0_ref, wh1_ref, bz_ref, bh_ref,
                wl_ref, bl_ref, out_ref):
    xb = x_ref[...]

    def dot(w_ref):
        return jnp.dot(xb, w_ref[...], preferred_element_type=jnp.float32)

    z = jax.nn.sigmoid(dot(wz0_ref) + dot(wz1_ref) + bz_ref[...])
    t = jnp.tanh(dot(wh0_ref) + dot(wh1_ref) + bh_ref[...])
    g = jnp.maximum((1.0 - z) * t, 0.0)
    out_ref[...] = (jnp.dot(g, wl_ref[...], preferred_element_type=jnp.float32)
                    + bl_ref[...])


def _dense_stage(x, wz0, wz1, wh0, wh1, bz, bh, wlt, bl):
    grid = (N // BLK,)
    full = lambda i: (jnp.zeros_like(i), jnp.zeros_like(i))
    return pl.pallas_call(
        _dense_body,
        grid=grid,
        in_specs=[
            pl.BlockSpec((BLK, F_IN), lambda i: (i, jnp.zeros_like(i))),
            pl.BlockSpec((F_IN, H_DIM), full),
            pl.BlockSpec((F_IN, H_DIM), full),
            pl.BlockSpec((F_IN, H_DIM), full),
            pl.BlockSpec((F_IN, H_DIM), full),
            pl.BlockSpec((1, H_DIM), full),
            pl.BlockSpec((1, H_DIM), full),
            pl.BlockSpec((H_DIM, 1), full),
            pl.BlockSpec((1, 1), full),
        ],
        out_specs=pl.BlockSpec((BLK, 1), lambda i: (i, jnp.zeros_like(i))),
        out_shape=jax.ShapeDtypeStruct((N, 1), jnp.float32),
    )(x, wz0, wz1, wh0, wh1, bz, bh, wlt, bl)


def _segmean_body(h_hbm, ids_hbm, out_hbm, stage_s, stage_c, vals_v, ids_v,
                  acc_s, acc_c, red_s, red_c, gbuf_s, gbuf_c):
    sid = lax.axis_index("s")
    cid = lax.axis_index("c")
    base = sid * CHUNK
    zero = jnp.zeros((16,), jnp.float32)
    one = jnp.ones((16,), jnp.float32)
    lane16 = lax.iota(jnp.int32, 16)
    lane_off = lane16 * B_PAD
    col0 = lane16 * 0

    @pl.when(sid != 15)
    def _():
        pltpu.sync_copy(h_hbm.at[pl.ds(base, CHUNK)], vals_v)
        pltpu.sync_copy(ids_hbm.at[pl.ds(base, CHUNK)], ids_v)

    @pl.when(sid == 15)
    def _():
        pltpu.sync_copy(h_hbm.at[pl.ds(15 * CHUNK, TAIL)],
                        vals_v.at[pl.ds(0, TAIL)])
        pltpu.sync_copy(ids_hbm.at[pl.ds(15 * CHUNK, TAIL)],
                        ids_v.at[pl.ds(0, TAIL)])

    for j in range(ACC // 16):
        acc_s[pl.ds(j * 16, 16)] = zero
        acc_c[pl.ds(j * 16, 16)] = zero

    def step(j):
        ids = plsc.load_gather(ids_v, [j * 16 + lane16, col0])
        v = vals_v[pl.ds(j * 16, 16)]
        idx = lane_off + ids
        plsc.addupdate_scatter(acc_s, [idx], v)
        plsc.addupdate_scatter(acc_c, [idx], one)

    for j in range(TAIL // 16):
        step(j)

    @pl.when(sid != 15)
    def _():
        for j in range(TAIL // 16, CHUNK // 16):
            step(j)

    for j in range(B_PAD // 16):
        s = zero
        c = zero
        for i in range(16):
            s = s + acc_s[pl.ds(i * B_PAD + j * 16, 16)]
            c = c + acc_c[pl.ds(i * B_PAD + j * 16, 16)]
        red_s[pl.ds(j * 16, 16)] = s
        red_c[pl.ds(j * 16, 16)] = c

    @pl.when(cid == 0)
    def _():
        pltpu.sync_copy(red_s, stage_s.at[sid])
        pltpu.sync_copy(red_c, stage_c.at[sid])

    plsc.subcore_barrier()

    @pl.when(jnp.logical_and(sid == 0, cid == 0))
    def _():
        pltpu.sync_copy(stage_s, gbuf_s)
        pltpu.sync_copy(stage_c, gbuf_c)
        for j in range(B_PAD // 16):
            s = jnp.zeros((16,), jnp.float32)
            c = jnp.zeros((16,), jnp.float32)
            for i in range(16):
                s = s + gbuf_s[i, pl.ds(j * 16, 16)]
                c = c + gbuf_c[i, pl.ds(j * 16, 16)]
            red_s[pl.ds(j * 16, 16)] = s / jnp.maximum(c, 1.0)
        pltpu.sync_copy(red_s, out_hbm)


def _segmean_stage(h_flat, ids):
    mesh = plsc.VectorSubcoreMesh(core_axis_name="c", subcore_axis_name="s")
    fn = functools.partial(
        pl.kernel,
        mesh=mesh,
        compiler_params=pltpu.CompilerParams(needs_layout_passes=False),
        out_type=(jax.ShapeDtypeStruct((B_PAD,), jnp.float32),
                  jax.ShapeDtypeStruct((16, B_PAD), jnp.float32),
                  jax.ShapeDtypeStruct((16, B_PAD), jnp.float32)),
        scratch_types=[
            pltpu.VMEM((CHUNK,), jnp.float32),
            pltpu.VMEM((CHUNK, 2), jnp.int32),
            pltpu.VMEM((ACC,), jnp.float32),
            pltpu.VMEM((ACC,), jnp.float32),
            pltpu.VMEM((B_PAD,), jnp.float32),
            pltpu.VMEM((B_PAD,), jnp.float32),
            pltpu.VMEM((16, B_PAD), jnp.float32),
            pltpu.VMEM((16, B_PAD), jnp.float32),
        ],
    )(_segmean_body)
    return fn(h_flat, ids)[0]


def kernel(x, edge_index, edge_weight, batch, Wz0, Wz1, Wr0, Wr1, Wh0, Wh1,
           bz, br, bh, W_lin, b_lin):
    del edge_index, edge_weight, Wr0, Wr1, br
    x = x.astype(jnp.float32)
    bz2 = bz.astype(jnp.float32).reshape(1, H_DIM)
    bh2 = bh.astype(jnp.float32).reshape(1, H_DIM)
    wlt = W_lin.astype(jnp.float32).reshape(H_DIM, 1)
    bl2 = b_lin.astype(jnp.float32).reshape(1, 1)

    h = _dense_stage(x, Wz0.astype(jnp.float32), Wz1.astype(jnp.float32),
                     Wh0.astype(jnp.float32), Wh1.astype(jnp.float32),
                     bz2, bh2, wlt, bl2)
    h_flat = h.reshape(N)
    if batch.dtype == jnp.int64:
        ids2 = jax.lax.bitcast_convert_type(batch, jnp.int32)
    else:
        ids2 = jnp.stack([batch.astype(jnp.int32)] * 2, axis=1)
    res = _segmean_stage(h_flat, ids2)
    return res[:B].reshape(B, 1)


# speedup vs baseline: 1.2313x; 1.2313x over previous
import functools

import jax
import jax.numpy as jnp
from jax import lax
from jax.experimental import pallas as pl
from jax.experimental.pallas import tpu as pltpu
from jax.experimental.pallas import tpu_sc as plsc

N = 10000
F_IN = 128
H_DIM = 32
B = 100

CHUNK = 640
TAIL = N - 15 * CHUNK
B_PAD = 112
ACC = 16 * B_PAD
BLK = 1000


def _dense_body(x_ref, wz0_ref, wz1_ref, wh0_ref, wh1_ref, bz_ref, bh_ref,
                wl_ref, bl_ref, out_ref):
    xb = x_ref[...]

    def dot(w_ref):
        return jnp.dot(xb, w_ref[...], preferred_element_type=jnp.float32)

    z = jax.nn.sigmoid(dot(wz0_ref) + dot(wz1_ref) + bz_ref[...])
    t = jnp.tanh(dot(wh0_ref) + dot(wh1_ref) + bh_ref[...])
    g = jnp.maximum((1.0 - z) * t, 0.0)
    out_ref[...] = (jnp.dot(g, wl_ref[...], preferred_element_type=jnp.float32)
                    + bl_ref[...])


def _dense_stage(x, wz0, wz1, wh0, wh1, bz, bh, wlt, bl):
    grid = (N // BLK,)
    full = lambda i: (jnp.zeros_like(i), jnp.zeros_like(i))
    return pl.pallas_call(
        _dense_body,
        grid=grid,
        in_specs=[
            pl.BlockSpec((BLK, F_IN), lambda i: (i, jnp.zeros_like(i))),
            pl.BlockSpec((F_IN, H_DIM), full),
            pl.BlockSpec((F_IN, H_DIM), full),
            pl.BlockSpec((F_IN, H_DIM), full),
            pl.BlockSpec((F_IN, H_DIM), full),
            pl.BlockSpec((1, H_DIM), full),
            pl.BlockSpec((1, H_DIM), full),
            pl.BlockSpec((H_DIM, 1), full),
            pl.BlockSpec((1, 1), full),
        ],
        out_specs=pl.BlockSpec((BLK, 1), lambda i: (i, jnp.zeros_like(i))),
        out_shape=jax.ShapeDtypeStruct((N, 1), jnp.float32),
    )(x, wz0, wz1, wh0, wh1, bz, bh, wlt, bl)


def _segmean_body(h_hbm, ids_hbm, out_hbm, stage_s, stage_c, vals_v, ids_v,
                  acc_s, acc_c, red_s, red_c, gbuf_s, gbuf_c):
    sid = lax.axis_index("s")
    cid = lax.axis_index("c")
    base = sid * CHUNK
    zero = jnp.zeros((16,), jnp.float32)
    one = jnp.ones((16,), jnp.float32)
    lane16 = lax.iota(jnp.int32, 16)
    lane_off = lane16 * B_PAD
    col0 = lane16 * 0

    @pl.when(sid != 15)
    def _():
        pltpu.sync_copy(h_hbm.at[pl.ds(base, CHUNK)], vals_v)
        pltpu.sync_copy(ids_hbm.at[pl.ds(base, CHUNK)], ids_v)

    @pl.when(sid == 15)
    def _():
        pltpu.sync_copy(h_hbm.at[pl.ds(15 * CHUNK, TAIL)],
                        vals_v.at[pl.ds(0, TAIL)])
        pltpu.sync_copy(ids_hbm.at[pl.ds(15 * CHUNK, TAIL)],
                        ids_v.at[pl.ds(0, TAIL)])

    for j in range(ACC // 16):
        acc_s[pl.ds(j * 16, 16)] = zero
        acc_c[pl.ds(j * 16, 16)] = zero

    def step(j):
        ids = ids_v[pl.ds(j * 16, 16)]
        v = vals_v[pl.ds(j * 16, 16)]
        idx = lane_off + ids
        plsc.addupdate_scatter(acc_s, [idx], v)
        plsc.addupdate_scatter(acc_c, [idx], one)

    for j in range(TAIL // 16):
        step(j)

    @pl.when(sid != 15)
    def _():
        for j in range(TAIL // 16, CHUNK // 16):
            step(j)

    for j in range(B_PAD // 16):
        s = zero
        c = zero
        for i in range(16):
            s = s + acc_s[pl.ds(i * B_PAD + j * 16, 16)]
            c = c + acc_c[pl.ds(i * B_PAD + j * 16, 16)]
        red_s[pl.ds(j * 16, 16)] = s
        red_c[pl.ds(j * 16, 16)] = c

    @pl.when(cid == 0)
    def _():
        pltpu.sync_copy(red_s, stage_s.at[sid])
        pltpu.sync_copy(red_c, stage_c.at[sid])

    plsc.subcore_barrier()

    @pl.when(jnp.logical_and(sid == 0, cid == 0))
    def _():
        pltpu.sync_copy(stage_s, gbuf_s)
        pltpu.sync_copy(stage_c, gbuf_c)
        for j in range(B_PAD // 16):
            s = jnp.zeros((16,), jnp.float32)
            c = jnp.zeros((16,), jnp.float32)
            for i in range(16):
                s = s + gbuf_s[i, pl.ds(j * 16, 16)]
                c = c + gbuf_c[i, pl.ds(j * 16, 16)]
            red_s[pl.ds(j * 16, 16)] = s / jnp.maximum(c, 1.0)
        pltpu.sync_copy(red_s, out_hbm)


def _segmean_stage(h_flat, ids):
    mesh = plsc.VectorSubcoreMesh(core_axis_name="c", subcore_axis_name="s")
    fn = functools.partial(
        pl.kernel,
        mesh=mesh,
        compiler_params=pltpu.CompilerParams(needs_layout_passes=False),
        out_type=(jax.ShapeDtypeStruct((B_PAD,), jnp.float32),
                  jax.ShapeDtypeStruct((16, B_PAD), jnp.float32),
                  jax.ShapeDtypeStruct((16, B_PAD), jnp.float32)),
        scratch_types=[
            pltpu.VMEM((CHUNK,), jnp.float32),
            pltpu.VMEM((CHUNK,), jnp.int32),
            pltpu.VMEM((ACC,), jnp.float32),
            pltpu.VMEM((ACC,), jnp.float32),
            pltpu.VMEM((B_PAD,), jnp.float32),
            pltpu.VMEM((B_PAD,), jnp.float32),
            pltpu.VMEM((16, B_PAD), jnp.float32),
            pltpu.VMEM((16, B_PAD), jnp.float32),
        ],
    )(_segmean_body)
    return fn(h_flat, ids)[0]


def kernel(x, edge_index, edge_weight, batch, Wz0, Wz1, Wr0, Wr1, Wh0, Wh1,
           bz, br, bh, W_lin, b_lin):
    del edge_index, edge_weight, Wr0, Wr1, br
    x = x.astype(jnp.float32)
    bz2 = bz.astype(jnp.float32).reshape(1, H_DIM)
    bh2 = bh.astype(jnp.float32).reshape(1, H_DIM)
    wlt = W_lin.astype(jnp.float32).reshape(H_DIM, 1)
    bl2 = b_lin.astype(jnp.float32).reshape(1, 1)

    h = _dense_stage(x, Wz0.astype(jnp.float32), Wz1.astype(jnp.float32),
                     Wh0.astype(jnp.float32), Wh1.astype(jnp.float32),
                     bz2, bh2, wlt, bl2)
    h_flat = h.reshape(N)
    ids = batch.astype(jnp.int32)
    res = _segmean_stage(h_flat, ids)
    return res[:B].reshape(B, 1)


# speedup vs baseline: 1.2888x; 1.0467x over previous
import functools

import jax
import jax.numpy as jnp
from jax import lax
from jax.experimental import pallas as pl
from jax.experimental.pallas import tpu as pltpu
from jax.experimental.pallas import tpu_sc as plsc

N = 10000
F_IN = 128
H_DIM = 32
B = 100

CHUNK = 640
TAIL = N - 15 * CHUNK
B_PAD = 112
ACC = 16 * B_PAD
BLK = 5000


def _dense_body(x_ref, wz0_ref, wz1_ref, wh0_ref, wh1_ref, bz_ref, bh_ref,
                wl_ref, bl_ref, out_ref):
    xb = x_ref[...]

    def dot(w_ref):
        return jnp.dot(xb, w_ref[...], preferred_element_type=jnp.float32)

    z = jax.nn.sigmoid(dot(wz0_ref) + dot(wz1_ref) + bz_ref[...])
    t = jnp.tanh(dot(wh0_ref) + dot(wh1_ref) + bh_ref[...])
    g = jnp.maximum((1.0 - z) * t, 0.0)
    out_ref[...] = (jnp.dot(g, wl_ref[...], preferred_element_type=jnp.float32)
                    + bl_ref[...])


def _dense_stage(x, wz0, wz1, wh0, wh1, bz, bh, wlt, bl):
    grid = (N // BLK,)
    full = lambda i: (jnp.zeros_like(i), jnp.zeros_like(i))
    return pl.pallas_call(
        _dense_body,
        grid=grid,
        in_specs=[
            pl.BlockSpec((BLK, F_IN), lambda i: (i, jnp.zeros_like(i))),
            pl.BlockSpec((F_IN, H_DIM), full),
            pl.BlockSpec((F_IN, H_DIM), full),
            pl.BlockSpec((F_IN, H_DIM), full),
            pl.BlockSpec((F_IN, H_DIM), full),
            pl.BlockSpec((1, H_DIM), full),
            pl.BlockSpec((1, H_DIM), full),
            pl.BlockSpec((H_DIM, 1), full),
            pl.BlockSpec((1, 1), full),
        ],
        out_specs=pl.BlockSpec((BLK, 1), lambda i: (i, jnp.zeros_like(i))),
        out_shape=jax.ShapeDtypeStruct((N, 1), jnp.float32),
    )(x, wz0, wz1, wh0, wh1, bz, bh, wlt, bl)


def _segmean_body(h_hbm, ids_hbm, out_hbm, stage_s, stage_c, vals_v, ids_v,
                  acc_s, acc_c, red_s, red_c, gbuf_s, gbuf_c):
    sid = lax.axis_index("s")
    cid = lax.axis_index("c")
    base = sid * CHUNK
    zero = jnp.zeros((16,), jnp.float32)
    one = jnp.ones((16,), jnp.float32)
    lane16 = lax.iota(jnp.int32, 16)
    lane_off = lane16 * B_PAD
    col0 = lane16 * 0

    @pl.when(sid != 15)
    def _():
        pltpu.sync_copy(h_hbm.at[pl.ds(base, CHUNK)], vals_v)
        pltpu.sync_copy(ids_hbm.at[pl.ds(base, CHUNK)], ids_v)

    @pl.when(sid == 15)
    def _():
        pltpu.sync_copy(h_hbm.at[pl.ds(15 * CHUNK, TAIL)],
                        vals_v.at[pl.ds(0, TAIL)])
        pltpu.sync_copy(ids_hbm.at[pl.ds(15 * CHUNK, TAIL)],
                        ids_v.at[pl.ds(0, TAIL)])

    for j in range(ACC // 16):
        acc_s[pl.ds(j * 16, 16)] = zero
        acc_c[pl.ds(j * 16, 16)] = zero

    def step(j):
        ids = ids_v[pl.ds(j * 16, 16)]
        v = vals_v[pl.ds(j * 16, 16)]
        idx = lane_off + ids
        plsc.addupdate_scatter(acc_s, [idx], v)
        plsc.addupdate_scatter(acc_c, [idx], one)

    for j in range(TAIL // 16):
        step(j)

    @pl.when(sid != 15)
    def _():
        for j in range(TAIL // 16, CHUNK // 16):
            step(j)

    for j in range(B_PAD // 16):
        s = zero
        c = zero
        for i in range(16):
            s = s + acc_s[pl.ds(i * B_PAD + j * 16, 16)]
            c = c + acc_c[pl.ds(i * B_PAD + j * 16, 16)]
        red_s[pl.ds(j * 16, 16)] = s
        red_c[pl.ds(j * 16, 16)] = c

    @pl.when(cid == 0)
    def _():
        pltpu.sync_copy(red_s, stage_s.at[sid])
        pltpu.sync_copy(red_c, stage_c.at[sid])

    plsc.subcore_barrier()

    @pl.when(jnp.logical_and(sid == 0, cid == 0))
    def _():
        pltpu.sync_copy(stage_s, gbuf_s)
        pltpu.sync_copy(stage_c, gbuf_c)
        for j in range(B_PAD // 16):
            s = jnp.zeros((16,), jnp.float32)
            c = jnp.zeros((16,), jnp.float32)
            for i in range(16):
                s = s + gbuf_s[i, pl.ds(j * 16, 16)]
                c = c + gbuf_c[i, pl.ds(j * 16, 16)]
            red_s[pl.ds(j * 16, 16)] = s / jnp.maximum(c, 1.0)
        pltpu.sync_copy(red_s, out_hbm)


def _segmean_stage(h_flat, ids):
    mesh = plsc.VectorSubcoreMesh(core_axis_name="c", subcore_axis_name="s")
    fn = functools.partial(
        pl.kernel,
        mesh=mesh,
        compiler_params=pltpu.CompilerParams(needs_layout_passes=False),
        out_type=(jax.ShapeDtypeStruct((B_PAD,), jnp.float32),
                  jax.ShapeDtypeStruct((16, B_PAD), jnp.float32),
                  jax.ShapeDtypeStruct((16, B_PAD), jnp.float32)),
        scratch_types=[
            pltpu.VMEM((CHUNK,), jnp.float32),
            pltpu.VMEM((CHUNK,), jnp.int32),
            pltpu.VMEM((ACC,), jnp.float32),
            pltpu.VMEM((ACC,), jnp.float32),
            pltpu.VMEM((B_PAD,), jnp.float32),
            pltpu.VMEM((B_PAD,), jnp.float32),
            pltpu.VMEM((16, B_PAD), jnp.float32),
            pltpu.VMEM((16, B_PAD), jnp.float32),
        ],
    )(_segmean_body)
    return fn(h_flat, ids)[0]


def kernel(x, edge_index, edge_weight, batch, Wz0, Wz1, Wr0, Wr1, Wh0, Wh1,
           bz, br, bh, W_lin, b_lin):
    del edge_index, edge_weight, Wr0, Wr1, br
    x = x.astype(jnp.float32)
    bz2 = bz.astype(jnp.float32).reshape(1, H_DIM)
    bh2 = bh.astype(jnp.float32).reshape(1, H_DIM)
    wlt = W_lin.astype(jnp.float32).reshape(H_DIM, 1)
    bl2 = b_lin.astype(jnp.float32).reshape(1, 1)

    h = _dense_stage(x, Wz0.astype(jnp.float32), Wz1.astype(jnp.float32),
                     Wh0.astype(jnp.float32), Wh1.astype(jnp.float32),
                     bz2, bh2, wlt, bl2)
    h_flat = h.reshape(N)
    ids = batch.astype(jnp.int32)
    res = _segmean_stage(h_flat, ids)
    return res[:B].reshape(B, 1)


# speedup vs baseline: 1.3720x; 1.0646x over previous
import functools

import jax
import jax.numpy as jnp
from jax import lax
from jax.experimental import pallas as pl
from jax.experimental.pallas import tpu as pltpu
from jax.experimental.pallas import tpu_sc as plsc

N = 10000
F_IN = 128
H_DIM = 32
B = 100

CHUNK = 640
TAIL = N - 15 * CHUNK
B_PAD = 112
ACC = 16 * B_PAD
BLK = 2000


def _dense_body(x_ref, wz0_ref, wz1_ref, wh0_ref, wh1_ref, bz_ref, bh_ref,
                wl_ref, bl_ref, out_ref):
    xb = x_ref[...]

    def dot(w_ref):
        return jnp.dot(xb, w_ref[...], preferred_element_type=jnp.float32)

    z = jax.nn.sigmoid(dot(wz0_ref) + dot(wz1_ref) + bz_ref[...])
    t = jnp.tanh(dot(wh0_ref) + dot(wh1_ref) + bh_ref[...])
    g = jnp.maximum((1.0 - z) * t, 0.0)
    out_ref[...] = (jnp.dot(g, wl_ref[...], preferred_element_type=jnp.float32)
                    + bl_ref[...])


def _dense_stage(x, wz0, wz1, wh0, wh1, bz, bh, wlt, bl):
    grid = (N // BLK,)
    full = lambda i: (jnp.zeros_like(i), jnp.zeros_like(i))
    return pl.pallas_call(
        _dense_body,
        grid=grid,
        in_specs=[
            pl.BlockSpec((BLK, F_IN), lambda i: (i, jnp.zeros_like(i))),
            pl.BlockSpec((F_IN, H_DIM), full),
            pl.BlockSpec((F_IN, H_DIM), full),
            pl.BlockSpec((F_IN, H_DIM), full),
            pl.BlockSpec((F_IN, H_DIM), full),
            pl.BlockSpec((1, H_DIM), full),
            pl.BlockSpec((1, H_DIM), full),
            pl.BlockSpec((H_DIM, 1), full),
            pl.BlockSpec((1, 1), full),
        ],
        out_specs=pl.BlockSpec((BLK, 1), lambda i: (i, jnp.zeros_like(i))),
        out_shape=jax.ShapeDtypeStruct((N, 1), jnp.float32),
    )(x, wz0, wz1, wh0, wh1, bz, bh, wlt, bl)


def _segmean_body(h_hbm, ids_hbm, out_hbm, stage_s, vals_v, ids_v,
                  acc_s, acc_c, red_s, red_c, gbuf_s):
    sid = lax.axis_index("s")
    cid = lax.axis_index("c")
    base = sid * CHUNK
    zero = jnp.zeros((16,), jnp.float32)
    one = jnp.ones((16,), jnp.float32)
    lane16 = lax.iota(jnp.int32, 16)
    lane_off = lane16 * B_PAD
    col0 = lane16 * 0

    @pl.when(sid != 15)
    def _():
        pltpu.sync_copy(h_hbm.at[pl.ds(base, CHUNK)], vals_v)
        pltpu.sync_copy(ids_hbm.at[pl.ds(base, CHUNK)], ids_v)

    @pl.when(sid == 15)
    def _():
        pltpu.sync_copy(h_hbm.at[pl.ds(15 * CHUNK, TAIL)],
                        vals_v.at[pl.ds(0, TAIL)])
        pltpu.sync_copy(ids_hbm.at[pl.ds(15 * CHUNK, TAIL)],
                        ids_v.at[pl.ds(0, TAIL)])

    for j in range(ACC // 16):
        acc_s[pl.ds(j * 16, 16)] = zero
        acc_c[pl.ds(j * 16, 16)] = zero

    def step(j):
        ids = ids_v[pl.ds(j * 16, 16)]
        v = vals_v[pl.ds(j * 16, 16)]
        idx = lane_off + ids
        plsc.addupdate_scatter(acc_s, [idx], v)
        plsc.addupdate_scatter(acc_c, [idx], one)

    for j in range(TAIL // 16):
        step(j)

    @pl.when(sid != 15)
    def _():
        for j in range(TAIL // 16, CHUNK // 16):
            step(j)

    for j in range(B_PAD // 16):
        s = zero
        c = zero
        for i in range(16):
            s = s + acc_s[pl.ds(i * B_PAD + j * 16, 16)]
            c = c + acc_c[pl.ds(i * B_PAD + j * 16, 16)]
        red_s[pl.ds(j * 16, 16)] = s
        red_s[pl.ds(B_PAD + j * 16, 16)] = c

    @pl.when(cid == 0)
    def _():
        pltpu.sync_copy(red_s, stage_s.at[sid])

    plsc.subcore_barrier()

    @pl.when(jnp.logical_and(sid < B_PAD // 16, cid == 0))
    def _():
        pltpu.sync_copy(stage_s, gbuf_s)
        s = jnp.zeros((16,), jnp.float32)
        c = jnp.zeros((16,), jnp.float32)
        off = sid * 16
        coff = B_PAD + sid * 16
        for i in range(16):
            s = s + gbuf_s[i, pl.ds(off, 16)]
            c = c + gbuf_s[i, pl.ds(coff, 16)]
        red_c[pl.ds(0, 16)] = s / jnp.maximum(c, 1.0)
        pltpu.sync_copy(red_c.at[pl.ds(0, 16)], out_hbm.at[pl.ds(off, 16)])


def _segmean_stage(h_flat, ids):
    mesh = plsc.VectorSubcoreMesh(core_axis_name="c", subcore_axis_name="s")
    fn = functools.partial(
        pl.kernel,
        mesh=mesh,
        compiler_params=pltpu.CompilerParams(needs_layout_passes=False),
        out_type=(jax.ShapeDtypeStruct((B_PAD,), jnp.float32),
                  jax.ShapeDtypeStruct((16, 2 * B_PAD), jnp.float32)),
        scratch_types=[
            pltpu.VMEM((CHUNK,), jnp.float32),
            pltpu.VMEM((CHUNK,), jnp.int32),
            pltpu.VMEM((ACC,), jnp.float32),
            pltpu.VMEM((ACC,), jnp.float32),
            pltpu.VMEM((2 * B_PAD,), jnp.float32),
            pltpu.VMEM((B_PAD,), jnp.float32),
            pltpu.VMEM((16, 2 * B_PAD), jnp.float32),
        ],
    )(_segmean_body)
    return fn(h_flat, ids)[0]


def kernel(x, edge_index, edge_weight, batch, Wz0, Wz1, Wr0, Wr1, Wh0, Wh1,
           bz, br, bh, W_lin, b_lin):
    del edge_index, edge_weight, Wr0, Wr1, br
    x = x.astype(jnp.float32)
    bz2 = bz.astype(jnp.float32).reshape(1, H_DIM)
    bh2 = bh.astype(jnp.float32).reshape(1, H_DIM)
    wlt = W_lin.astype(jnp.float32).reshape(H_DIM, 1)
    bl2 = b_lin.astype(jnp.float32).reshape(1, 1)

    h = _dense_stage(x, Wz0.astype(jnp.float32), Wz1.astype(jnp.float32),
                     Wh0.astype(jnp.float32), Wh1.astype(jnp.float32),
                     bz2, bh2, wlt, bl2)
    h_flat = h.reshape(N)
    ids = batch.astype(jnp.int32)
    res = _segmean_stage(h_flat, ids)
    return res[:B].reshape(B, 1)
